# baseline (speedup 1.0000x reference)
import functools

import jax
import jax.numpy as jnp
from jax.experimental import pallas as pl
from jax.experimental.pallas import tpu as pltpu

N = 10000
NFEAT = 128
NHID = 128
NCLASS = 64
BLK_R = 400
HALF = BLK_R // 2


def _body(adjA_ref, adjB_ref, x_ref, w1a_ref, w1b_ref, w2a_ref, w2b_ref,
          w3_ref, b3_ref, out_ref, h_ref):
    p = pl.program_id(0)
    i = pl.program_id(1)

    def layer1(adjblk, row0):
        rowsum = jnp.sum(adjblk, axis=1, keepdims=True)
        inv = 1.0 / (rowsum + 1.0)
        neigh = jnp.dot(adjblk, x_ref[...], preferred_element_type=jnp.float32) * inv
        xblk = x_ref[pl.ds(row0, HALF), :]
        pre = (jnp.dot(xblk, w1a_ref[...], preferred_element_type=jnp.float32)
               + jnp.dot(neigh, w1b_ref[...], preferred_element_type=jnp.float32))
        h_ref[pl.ds(row0, HALF), :] = jnp.maximum(pre, 0.0)

    def layer2(adjblk, row0, off):
        rowsum = jnp.sum(adjblk, axis=1, keepdims=True)
        inv = 1.0 / (rowsum + 1.0)
        neigh = jnp.dot(adjblk, h_ref[...], preferred_element_type=jnp.float32) * inv
        hblk = h_ref[pl.ds(row0, HALF), :]
        h2 = jnp.maximum(
            jnp.dot(hblk, w2a_ref[...], preferred_element_type=jnp.float32)
            + jnp.dot(neigh, w2b_ref[...], preferred_element_type=jnp.float32),
            0.0)
        logits = (jnp.dot(h2, w3_ref[...], preferred_element_type=jnp.float32)
                  + b3_ref[...])
        m = jnp.max(logits, axis=1, keepdims=True)
        lse = m + jnp.log(jnp.sum(jnp.exp(logits - m), axis=1, keepdims=True))
        out_ref[pl.ds(off, HALF), :] = logits - lse

    @pl.when(p == 0)
    def _phase0():
        layer1(adjA_ref[...], i * BLK_R)
        layer1(adjB_ref[...], i * BLK_R + HALF)

    @pl.when(p == 1)
    def _phase1():
        layer2(adjA_ref[...], i * BLK_R, 0)
        layer2(adjB_ref[...], i * BLK_R + HALF, HALF)


@functools.partial(jax.jit, static_argnames=("interpret",))
def kernel(x, adj, W1, W2, W3, b3, interpret=False):
    w1a = W1[:, :NFEAT].T
    w1b = W1[:, NFEAT:].T
    w2a = W2[:, :NHID].T
    w2b = W2[:, NHID:].T
    w3 = W3.T
    b3r = b3.reshape(1, NCLASS)

    grid = (2, N // BLK_R)
    adjA_spec = pl.BlockSpec((HALF, N), lambda p, i: (2 * i, 0))
    adjB_spec = pl.BlockSpec((HALF, N), lambda p, i: (2 * i + 1, 0))
    x_spec = pl.BlockSpec((N, NFEAT), lambda p, i: (0, 0))
    w_spec = pl.BlockSpec((NFEAT, NHID), lambda p, i: (0, 0))

    out = pl.pallas_call(
        _body,
        grid=grid,
        in_specs=[
            adjA_spec, adjB_spec,
            x_spec,
            w_spec, w_spec, w_spec, w_spec,
            pl.BlockSpec((NHID, NCLASS), lambda p, i: (0, 0)),
            pl.BlockSpec((1, NCLASS), lambda p, i: (0, 0)),
        ],
        out_specs=pl.BlockSpec((BLK_R, NCLASS), lambda p, i: (i, 0)),
        out_shape=jax.ShapeDtypeStruct((N, NCLASS), jnp.float32),
        scratch_shapes=[pltpu.VMEM((N, NHID), jnp.float32)],
        interpret=interpret,
    )(adj, adj, x, w1a, w1b, w2a, w2b, w3, b3r)
    return out
